# speedup vs baseline: 1.5457x; 1.5457x over previous
import jax
import jax.numpy as jnp
from jax.experimental import pallas as pl


def kernel(x_ids, table):
    bsz, seq_len = x_ids.shape
    d = table.shape[1]
    BT = 512
    nb = seq_len // BT

    def body(tab_ref, out_ref):
        out_ref[...] = tab_ref[...][None]

    out = pl.pallas_call(
        body,
        grid=(nb, bsz),
        in_specs=[pl.BlockSpec((BT, d), lambda j, b: (j, 0))],
        out_specs=pl.BlockSpec((1, BT, d), lambda j, b: (b, j, 0)),
        out_shape=jax.ShapeDtypeStruct((bsz, seq_len, d), table.dtype),
    )(table)
    return out


# speedup vs baseline: 2.3633x; 1.5289x over previous
import jax
import jax.numpy as jnp
from jax.experimental import pallas as pl


def kernel(x_ids, table):
    bsz, seq_len = x_ids.shape
    d = table.shape[1]
    BT = 1024
    nb = seq_len // BT

    def body(tab_ref, out_ref):
        out_ref[...] = jnp.broadcast_to(tab_ref[...][None], out_ref.shape)

    out = pl.pallas_call(
        body,
        grid=(nb,),
        in_specs=[pl.BlockSpec((BT, d), lambda j: (j, 0))],
        out_specs=pl.BlockSpec((bsz, BT, d), lambda j: (0, j, 0)),
        out_shape=jax.ShapeDtypeStruct((bsz, seq_len, d), table.dtype),
    )(table)
    return out
